# baseline (speedup 1.0000x reference)
import jax
import jax.numpy as jnp
from jax import lax
from jax.experimental import pallas as pl
from jax.experimental.pallas import tpu as pltpu
from jax.experimental.pallas import tpu_sc as plsc

VOCAB = 1000
SEQ = 50
NBATCH = 1024
NC, NS, L = 2, 16, 16
NW = NC * NS
BATCH_PER_W = NBATCH // NW
ROWS_PER_W = BATCH_PER_W * SEQ
NSLOT = 2
IDX_PAD = ROWS_PER_W + L


def _sc_body(x_hbm, o_hbm, idx_v, buf, sems):
    wid = lax.axis_index("s") * NC + lax.axis_index("c")
    base = wid * BATCH_PER_W

    pltpu.sync_copy(x_hbm.at[pl.ds(base * SEQ, ROWS_PER_W)], idx_v.at[pl.ds(0, ROWS_PER_W)])

    zeros16 = jnp.zeros((L,), jnp.float32)
    ones16 = jnp.full((L,), 1.0, jnp.float32)
    row_iota = lax.broadcasted_iota(jnp.int32, (L,), 0)

    @pl.loop(0, NSLOT * SEQ)
    def _zrow(r):
        s = r // SEQ
        j = r - s * SEQ

        @pl.loop(0, VOCAB - L + 1, step=L)
        def _zcol(c):
            buf[s, j, pl.ds(c, L)] = zeros16

        buf[s, j, pl.ds(VOCAB - L, L)] = zeros16

    def _scatter(slot, g, values16):
        for k in range(4):
            rows16 = row_iota + (k * L)
            cols16 = idx_v[pl.ds(g * SEQ + k * L, L)]
            if (k + 1) * L <= SEQ:
                plsc.store_scatter(buf.at[slot], [rows16, cols16], values16)
            else:
                mask = rows16 < SEQ
                plsc.store_scatter(buf.at[slot], [rows16, cols16], values16, mask=mask)

    @pl.loop(0, BATCH_PER_W, step=NSLOT)
    def _group(g0):
        for b in range(NSLOT):
            g = g0 + b

            @pl.when(g0 >= NSLOT)
            def _recycle(b=b, g=g):
                gp = g - NSLOT
                pltpu.make_async_copy(
                    buf.at[b], o_hbm.at[base + gp], sems.at[b]
                ).wait()
                _scatter(b, gp, zeros16)

            _scatter(b, g, ones16)
            pltpu.make_async_copy(buf.at[b], o_hbm.at[base + g], sems.at[b]).start()

    for b in range(NSLOT):
        g = BATCH_PER_W - NSLOT + b
        pltpu.make_async_copy(buf.at[b], o_hbm.at[base + g], sems.at[b]).wait()


def kernel(x):
    B, S = x.shape
    xf = x.astype(jnp.int32).reshape(B * S)
    mesh = plsc.VectorSubcoreMesh(core_axis_name="c", subcore_axis_name="s")
    sc = pl.kernel(
        _sc_body,
        out_type=jax.ShapeDtypeStruct((B, S, VOCAB), jnp.float32),
        mesh=mesh,
        compiler_params=pltpu.CompilerParams(
            use_tc_tiling_on_sc=False, needs_layout_passes=False
        ),
        scratch_types=[
            pltpu.VMEM((IDX_PAD,), jnp.int32),
            pltpu.VMEM((NSLOT, SEQ, VOCAB), jnp.float32),
            pltpu.SemaphoreType.DMA((NSLOT,)),
        ],
    )
    return sc(xf)
